# baseline (speedup 1.0000x reference)
import functools

import jax
import jax.numpy as jnp
from jax import lax
from jax.experimental import pallas as pl
from jax.experimental.pallas import tpu as pltpu

_TV = 2048
_NBUF = 4
_NCHUNK = 4


def _fused_body(
    ids_ref,
    w_ref,
    b_ref,
    e_ref,
    o_hbm,
    x_hbm,
    bufs,
    xbufs,
    sems,
    xsem,
    *,
    nv,
    nv_main,
    tv,
    Bc,
    nchunk,
    V,
):
    c = pl.program_id(0)
    j = pl.program_id(1)

    @pl.when(c < nchunk)
    def _():
        e16 = e_ref[...].astype(jnp.bfloat16)
        lane = lax.broadcasted_iota(jnp.int32, e16.shape, 1)
        e16m = jnp.where(lane < V - j * tv, e16, jnp.bfloat16(0))
        rows = lax.broadcasted_iota(jnp.int32, (tv, Bc), 0) + j * tv
        for k in range(nchunk):

            @pl.when(c == k)
            def _(k=k):
                ids2 = ids_ref[:, pl.ds(k * Bc, Bc)]
                oh = (rows == jnp.broadcast_to(ids2, (tv, Bc))).astype(
                    jnp.bfloat16
                )
                part = lax.dot_general(
                    e16m,
                    oh,
                    (((1,), (0,)), ((), ())),
                    preferred_element_type=jnp.float32,
                )

                @pl.when(j == 0)
                def _():
                    xbufs[k][...] = part

                @pl.when(j > 0)
                def _():
                    xbufs[k][...] = xbufs[k][...] + part

                @pl.when(j == nv - 1)
                def _():
                    pltpu.make_async_copy(
                        xbufs[k], x_hbm.at[:, pl.ds(k * Bc, Bc)], xsem
                    ).start()

    @pl.when(jnp.logical_and(c >= 1, j < nv_main))
    def _():
        ch = c - 1
        s = ch * nv_main + j
        slot = lax.rem(s, _NBUF)
        wv = w_ref[:, pl.ds(j * tv, tv)]
        bv = b_ref[:, pl.ds(j * tv, tv)]
        wa = jnp.concatenate([wv, bv], axis=0)
        for k in range(nchunk):

            @pl.when(ch == k)
            def _(k=k):
                xa = jnp.concatenate(
                    [xbufs[k][...], jnp.ones((1, Bc), jnp.float32)], axis=0
                )
                ot = lax.dot_general(
                    wa,
                    xa,
                    (((0,), (0,)), ((), ())),
                    preferred_element_type=jnp.float32,
                )
                for q in range(_NBUF):

                    @pl.when(slot == q)
                    def _(q=q):
                        @pl.when(s >= _NBUF)
                        def _():
                            ps = s - _NBUF
                            pj = lax.rem(ps, nv_main)
                            pch = ps // nv_main
                            pltpu.make_async_copy(
                                bufs[q],
                                o_hbm.at[
                                    pl.ds(pj * tv, tv), pl.ds(pch * Bc, Bc)
                                ],
                                sems.at[q],
                            ).wait()

                        bufs[q][...] = ot
                        pltpu.make_async_copy(
                            bufs[q],
                            o_hbm.at[pl.ds(j * tv, tv), pl.ds(ch * Bc, Bc)],
                            sems.at[q],
                        ).start()

    @pl.when(jnp.logical_and(c == nchunk, j == nv - 1))
    def _():
        total = nchunk * nv_main
        for ps in range(total - _NBUF, total):
            q = ps % _NBUF
            pj = ps % nv_main
            pch = ps // nv_main
            pltpu.make_async_copy(
                bufs[q],
                o_hbm.at[pl.ds(pj * tv, tv), pl.ds(pch * Bc, Bc)],
                sems.at[q],
            ).wait()
        for k in range(nchunk):
            pltpu.make_async_copy(
                xbufs[k], x_hbm.at[:, pl.ds(k * Bc, Bc)], xsem
            ).wait()


def _tail_body(o_in, x_ref, w_ref, b_ref, o_ref):
    xa = jnp.concatenate(
        [x_ref[...], jnp.ones((1, x_ref.shape[1]), jnp.float32)], axis=0
    )
    wa = jnp.concatenate([w_ref[...], b_ref[...]], axis=0)
    o_ref[...] = lax.dot_general(
        wa,
        xa,
        (((0,), (0,)), ((), ())),
        preferred_element_type=jnp.float32,
    )


def kernel(input_ids, embedding, W, b):
    (B,) = input_ids.shape
    V, H = embedding.shape
    tv = _TV
    nv = pl.cdiv(V, tv)
    nv_main = V // tv
    nchunk = _NCHUNK
    Bc = B // nchunk
    ids2 = input_ids.astype(jnp.int32).reshape(1, B)
    wT = W.T
    embT = embedding.T
    b2 = b.reshape(1, V)

    body = functools.partial(
        _fused_body, nv=nv, nv_main=nv_main, tv=tv, Bc=Bc, nchunk=nchunk, V=V
    )
    outT, xT = pl.pallas_call(
        body,
        grid=(nchunk + 1, nv),
        in_specs=[
            pl.BlockSpec((1, B), lambda c, j: (0, 0)),
            pl.BlockSpec((H, V), lambda c, j: (0, 0)),
            pl.BlockSpec((1, V), lambda c, j: (0, 0)),
            pl.BlockSpec(
                (H, tv),
                lambda c, j: (0, jnp.where(c < _NCHUNK, j, 0)),
            ),
        ],
        out_specs=[
            pl.BlockSpec(memory_space=pl.ANY),
            pl.BlockSpec(memory_space=pl.ANY),
        ],
        out_shape=[
            jax.ShapeDtypeStruct((V, B), jnp.float32),
            jax.ShapeDtypeStruct((H, B), jnp.float32),
        ],
        scratch_shapes=[
            [pltpu.VMEM((tv, Bc), jnp.float32) for _ in range(_NBUF)],
            [pltpu.VMEM((H, Bc), jnp.float32) for _ in range(nchunk)],
            pltpu.SemaphoreType.DMA((_NBUF,)),
            pltpu.SemaphoreType.DMA,
        ],
        compiler_params=pltpu.CompilerParams(
            dimension_semantics=("arbitrary", "arbitrary"),
            vmem_limit_bytes=100 * 1024 * 1024,
        ),
    )(ids2, wT, b2, embT)

    jt = nv_main
    outT = pl.pallas_call(
        _tail_body,
        grid=(1,),
        in_specs=[
            pl.BlockSpec(memory_space=pl.ANY),
            pl.BlockSpec((H, B), lambda i: (0, 0)),
            pl.BlockSpec((H, tv), lambda i: (0, jt)),
            pl.BlockSpec((1, tv), lambda i: (0, jt)),
        ],
        out_specs=pl.BlockSpec((tv, B), lambda i: (jt, 0)),
        out_shape=jax.ShapeDtypeStruct((V, B), jnp.float32),
        input_output_aliases={0: 0},
    )(outT, xT, wT, b2)
    return outT.T


# speedup vs baseline: 1.3606x; 1.3606x over previous
import functools

import jax
import jax.numpy as jnp
from jax import lax
from jax.experimental import pallas as pl
from jax.experimental.pallas import tpu as pltpu

_TV = 2048
_NBUF = 4
_NCHUNK = 2


def _fused_body(
    ids_ref,
    w_ref,
    b_ref,
    iota_ref,
    e_ref,
    o_hbm,
    x_hbm,
    bufs,
    xbufs,
    sems,
    xsem,
    *,
    nv,
    nv_main,
    tv,
    Bc,
    nchunk,
    V,
):
    c = pl.program_id(0)
    j = pl.program_id(1)

    @pl.when(c < nchunk)
    def _():
        e16 = e_ref[...].astype(jnp.bfloat16)
        lane = lax.broadcasted_iota(jnp.int32, e16.shape, 1)
        e16m = jnp.where(lane < V - j * tv, e16, jnp.bfloat16(0))
        ioc = jnp.broadcast_to(iota_ref[...], (tv, Bc))
        for k in range(nchunk):

            @pl.when(c == k)
            def _(k=k):
                ids_rel = ids_ref[:, pl.ds(k * Bc, Bc)] - j * tv
                oh = (ioc == jnp.broadcast_to(ids_rel, (tv, Bc))).astype(
                    jnp.bfloat16
                )
                part = lax.dot_general(
                    e16m,
                    oh,
                    (((1,), (0,)), ((), ())),
                    preferred_element_type=jnp.float32,
                )

                @pl.when(j == 0)
                def _():
                    xbufs[k][...] = part

                @pl.when(j > 0)
                def _():
                    xbufs[k][...] = xbufs[k][...] + part

                @pl.when(j == nv - 1)
                def _():
                    pltpu.make_async_copy(
                        xbufs[k], x_hbm.at[:, pl.ds(k * Bc, Bc)], xsem
                    ).start()

    @pl.when(jnp.logical_and(c >= 1, j < nv_main))
    def _():
        ch = c - 1
        s = ch * nv_main + j
        slot = lax.rem(s, _NBUF)
        wv = w_ref[:, pl.ds(j * tv, tv)]
        bv = b_ref[:, pl.ds(j * tv, tv)]
        wa = jnp.concatenate([wv, bv], axis=0)
        for k in range(nchunk):

            @pl.when(ch == k)
            def _(k=k):
                xa = jnp.concatenate(
                    [xbufs[k][...], jnp.ones((1, Bc), jnp.float32)], axis=0
                )
                ot = lax.dot_general(
                    wa,
                    xa,
                    (((0,), (0,)), ((), ())),
                    preferred_element_type=jnp.float32,
                )
                for q in range(_NBUF):

                    @pl.when(slot == q)
                    def _(q=q):
                        @pl.when(s >= _NBUF)
                        def _():
                            ps = s - _NBUF
                            pj = lax.rem(ps, nv_main)
                            pch = ps // nv_main
                            pltpu.make_async_copy(
                                bufs[q],
                                o_hbm.at[
                                    pl.ds(pj * tv, tv), pl.ds(pch * Bc, Bc)
                                ],
                                sems.at[q],
                            ).wait()

                        bufs[q][...] = ot
                        pltpu.make_async_copy(
                            bufs[q],
                            o_hbm.at[pl.ds(j * tv, tv), pl.ds(ch * Bc, Bc)],
                            sems.at[q],
                        ).start()

    @pl.when(jnp.logical_and(c == nchunk, j == nv - 1))
    def _():
        total = nchunk * nv_main
        for ps in range(total - _NBUF, total):
            q = ps % _NBUF
            pj = ps % nv_main
            pch = ps // nv_main
            pltpu.make_async_copy(
                bufs[q],
                o_hbm.at[pl.ds(pj * tv, tv), pl.ds(pch * Bc, Bc)],
                sems.at[q],
            ).wait()
        for k in range(nchunk):
            pltpu.make_async_copy(
                xbufs[k], x_hbm.at[:, pl.ds(k * Bc, Bc)], xsem
            ).wait()


def _tail_body(o_in, x_ref, w_ref, b_ref, o_ref):
    xa = jnp.concatenate(
        [x_ref[...], jnp.ones((1, x_ref.shape[1]), jnp.float32)], axis=0
    )
    wa = jnp.concatenate([w_ref[...], b_ref[...]], axis=0)
    o_ref[...] = lax.dot_general(
        wa,
        xa,
        (((0,), (0,)), ((), ())),
        preferred_element_type=jnp.float32,
    )


def kernel(input_ids, embedding, W, b):
    (B,) = input_ids.shape
    V, H = embedding.shape
    tv = _TV
    nv = pl.cdiv(V, tv)
    nv_main = V // tv
    nchunk = _NCHUNK
    Bc = B // nchunk
    ids2 = input_ids.astype(jnp.int32).reshape(1, B)
    wT = W.T
    embT = embedding.T
    b2 = b.reshape(1, V)

    body = functools.partial(
        _fused_body, nv=nv, nv_main=nv_main, tv=tv, Bc=Bc, nchunk=nchunk, V=V
    )
    outT, xT = pl.pallas_call(
        body,
        grid=(nchunk + 1, nv),
        in_specs=[
            pl.BlockSpec((1, B), lambda c, j: (0, 0)),
            pl.BlockSpec((H, V), lambda c, j: (0, 0)),
            pl.BlockSpec((1, V), lambda c, j: (0, 0)),
            pl.BlockSpec((tv, 1), lambda c, j: (0, 0)),
            pl.BlockSpec(
                (H, tv),
                lambda c, j: (0, jnp.where(c < _NCHUNK, j, 0)),
            ),
        ],
        out_specs=[
            pl.BlockSpec(memory_space=pl.ANY),
            pl.BlockSpec(memory_space=pl.ANY),
        ],
        out_shape=[
            jax.ShapeDtypeStruct((V, B), jnp.float32),
            jax.ShapeDtypeStruct((H, B), jnp.float32),
        ],
        scratch_shapes=[
            [pltpu.VMEM((tv, Bc), jnp.float32) for _ in range(_NBUF)],
            [pltpu.VMEM((H, Bc), jnp.float32) for _ in range(nchunk)],
            pltpu.SemaphoreType.DMA((_NBUF,)),
            pltpu.SemaphoreType.DMA,
        ],
        compiler_params=pltpu.CompilerParams(
            dimension_semantics=("arbitrary", "arbitrary"),
            vmem_limit_bytes=100 * 1024 * 1024,
        ),
    )(ids2, wT, b2, jnp.arange(tv, dtype=jnp.int32).reshape(tv, 1), embT)

    jt = nv_main
    outT = pl.pallas_call(
        _tail_body,
        grid=(1,),
        in_specs=[
            pl.BlockSpec(memory_space=pl.ANY),
            pl.BlockSpec((H, B), lambda i: (0, 0)),
            pl.BlockSpec((H, tv), lambda i: (0, jt)),
            pl.BlockSpec((1, tv), lambda i: (0, jt)),
        ],
        out_specs=pl.BlockSpec((tv, B), lambda i: (jt, 0)),
        out_shape=jax.ShapeDtypeStruct((V, B), jnp.float32),
        input_output_aliases={0: 0},
    )(outT, xT, wT, b2)
    return outT.T


# speedup vs baseline: 1.4554x; 1.0697x over previous
import functools

import jax
import jax.numpy as jnp
from jax import lax
from jax.experimental import pallas as pl
from jax.experimental.pallas import tpu as pltpu

_TV = 2048
_NBUF = 4
_NCHUNK = 2


def _fused_body(
    ids_ref,
    w_ref,
    b_ref,
    iota_ref,
    e_ref,
    o_hbm,
    x_hbm,
    bufs,
    xbufs,
    sems,
    xsem,
    *,
    nv,
    nv_main,
    tv,
    Bc,
    nchunk,
    V,
):
    c = pl.program_id(0)
    j = pl.program_id(1)

    @pl.when(c < nchunk)
    def _():
        e16 = e_ref[...].astype(jnp.bfloat16)
        lane = lax.broadcasted_iota(jnp.int32, e16.shape, 1)
        e16m = jnp.where(lane < V - j * tv, e16, jnp.bfloat16(0))
        ioc = lax.broadcasted_iota(jnp.int32, (tv, Bc), 0)
        for k in range(nchunk):

            @pl.when(c == k)
            def _(k=k):
                ids_rel = ids_ref[:, pl.ds(k * Bc, Bc)] - j * tv
                oh = (ioc == jnp.broadcast_to(ids_rel, (tv, Bc))).astype(
                    jnp.bfloat16
                )
                part = lax.dot_general(
                    e16m,
                    oh,
                    (((1,), (0,)), ((), ())),
                    preferred_element_type=jnp.float32,
                )

                @pl.when(j == 0)
                def _():
                    xbufs[k][...] = part

                @pl.when(j > 0)
                def _():
                    xbufs[k][...] = xbufs[k][...] + part

                @pl.when(j == nv - 1)
                def _():
                    pltpu.make_async_copy(
                        xbufs[k], x_hbm.at[:, pl.ds(k * Bc, Bc)], xsem
                    ).start()

    @pl.when(jnp.logical_and(c >= 1, j < nv_main))
    def _():
        ch = c - 1
        s = ch * nv_main + j
        slot = lax.rem(s, _NBUF)
        wv = w_ref[:, pl.ds(j * tv, tv)]
        bv = b_ref[:, pl.ds(j * tv, tv)]
        wa = jnp.concatenate([wv, bv], axis=0)
        for k in range(nchunk):

            @pl.when(ch == k)
            def _(k=k):
                xa = jnp.concatenate(
                    [xbufs[k][...], jnp.ones((1, Bc), jnp.float32)], axis=0
                )
                ot = lax.dot_general(
                    wa,
                    xa,
                    (((0,), (0,)), ((), ())),
                    preferred_element_type=jnp.float32,
                )
                for q in range(_NBUF):

                    @pl.when(slot == q)
                    def _(q=q):
                        @pl.when(s >= _NBUF)
                        def _():
                            ps = s - _NBUF
                            pj = lax.rem(ps, nv_main)
                            pch = ps // nv_main
                            pltpu.make_async_copy(
                                bufs[q],
                                o_hbm.at[
                                    pl.ds(pj * tv, tv), pl.ds(pch * Bc, Bc)
                                ],
                                sems.at[q],
                            ).wait()

                        bufs[q][...] = ot
                        pltpu.make_async_copy(
                            bufs[q],
                            o_hbm.at[pl.ds(j * tv, tv), pl.ds(ch * Bc, Bc)],
                            sems.at[q],
                        ).start()

    @pl.when(jnp.logical_and(c == nchunk, j == nv - 1))
    def _():
        total = nchunk * nv_main
        for ps in range(total - _NBUF, total):
            q = ps % _NBUF
            pj = ps % nv_main
            pch = ps // nv_main
            pltpu.make_async_copy(
                bufs[q],
                o_hbm.at[pl.ds(pj * tv, tv), pl.ds(pch * Bc, Bc)],
                sems.at[q],
            ).wait()
        for k in range(nchunk):
            pltpu.make_async_copy(
                xbufs[k], x_hbm.at[:, pl.ds(k * Bc, Bc)], xsem
            ).wait()


def _tail_body(o_in, x_ref, w_ref, b_ref, o_ref):
    xa = jnp.concatenate(
        [x_ref[...], jnp.ones((1, x_ref.shape[1]), jnp.float32)], axis=0
    )
    wa = jnp.concatenate([w_ref[...], b_ref[...]], axis=0)
    o_ref[...] = lax.dot_general(
        wa,
        xa,
        (((0,), (0,)), ((), ())),
        preferred_element_type=jnp.float32,
    )


def kernel(input_ids, embedding, W, b):
    (B,) = input_ids.shape
    V, H = embedding.shape
    tv = _TV
    nv = pl.cdiv(V, tv)
    nv_main = V // tv
    nchunk = _NCHUNK
    Bc = B // nchunk
    ids2 = input_ids.astype(jnp.int32).reshape(1, B)
    wT = W.T
    embT = embedding.T
    b2 = b.reshape(1, V)

    body = functools.partial(
        _fused_body, nv=nv, nv_main=nv_main, tv=tv, Bc=Bc, nchunk=nchunk, V=V
    )
    outT, xT = pl.pallas_call(
        body,
        grid=(nchunk + 1, nv),
        in_specs=[
            pl.BlockSpec((1, B), lambda c, j: (0, 0)),
            pl.BlockSpec((H, V), lambda c, j: (0, 0)),
            pl.BlockSpec((1, V), lambda c, j: (0, 0)),
            pl.BlockSpec((tv, 1), lambda c, j: (0, 0)),
            pl.BlockSpec(
                (H, tv),
                lambda c, j: (0, jnp.where(c < _NCHUNK, j, 0)),
            ),
        ],
        out_specs=[
            pl.BlockSpec(memory_space=pl.ANY),
            pl.BlockSpec(memory_space=pl.ANY),
        ],
        out_shape=[
            jax.ShapeDtypeStruct((V, B), jnp.float32),
            jax.ShapeDtypeStruct((H, B), jnp.float32),
        ],
        scratch_shapes=[
            [pltpu.VMEM((tv, Bc), jnp.float32) for _ in range(_NBUF)],
            [pltpu.VMEM((H, Bc), jnp.float32) for _ in range(nchunk)],
            pltpu.SemaphoreType.DMA((_NBUF,)),
            pltpu.SemaphoreType.DMA,
        ],
        compiler_params=pltpu.CompilerParams(
            dimension_semantics=("arbitrary", "arbitrary"),
            vmem_limit_bytes=100 * 1024 * 1024,
        ),
    )(ids2, wT, b2, jnp.arange(tv, dtype=jnp.int32).reshape(tv, 1), embT)

    jt = nv_main
    outT = pl.pallas_call(
        _tail_body,
        grid=(1,),
        in_specs=[
            pl.BlockSpec(memory_space=pl.ANY),
            pl.BlockSpec((H, B), lambda i: (0, 0)),
            pl.BlockSpec((H, tv), lambda i: (0, jt)),
            pl.BlockSpec((1, tv), lambda i: (0, jt)),
        ],
        out_specs=pl.BlockSpec((tv, B), lambda i: (jt, 0)),
        out_shape=jax.ShapeDtypeStruct((V, B), jnp.float32),
        input_output_aliases={0: 0},
    )(outT, xT, wT, b2)
    return outT.T
